# Initial kernel scaffold; baseline (speedup 1.0000x reference)
import jax
import jax.numpy as jnp
from jax.experimental import pallas as pl


def kernel(x, perm):
    raise NotImplementedError("write your pallas kernel here")


# speedup vs baseline: 2.7820x; 2.7820x over previous
import functools

import jax
import jax.numpy as jnp
from jax import lax
from jax.experimental import pallas as pl
from jax.experimental.pallas import tpu as pltpu
from jax.experimental.pallas import tpu_sc as plsc


def _gather_rows(xf, perm_i32, B, S, D):
    info = plsc.get_sparse_core_info()
    NC, NS, L = info.num_cores, info.num_subcores, info.num_lanes
    NW = NC * NS
    rows_total = B * S
    rows_per_w = rows_total // NW
    CH = 64
    n_ch = rows_per_w // CH

    mesh = plsc.VectorSubcoreMesh(core_axis_name="c", subcore_axis_name="s")

    @functools.partial(
        pl.kernel,
        mesh=mesh,
        out_type=jax.ShapeDtypeStruct((rows_total, D), jnp.float32),
        scratch_types=[
            pltpu.VMEM((rows_per_w,), jnp.int32),
            pltpu.VMEM((CH, D), jnp.float32),
            pltpu.SemaphoreType.DMA,
        ],
    )
    def _k(x_hbm, perm_hbm, out_hbm, idx_v, rows_v, sem):
        wid = lax.axis_index("s") * NC + lax.axis_index("c")
        base = wid * rows_per_w
        seq0 = base % S
        boff = base - seq0
        pltpu.sync_copy(perm_hbm.at[pl.ds(seq0, rows_per_w)], idx_v)
        for j in range(rows_per_w // L):
            sl = pl.ds(j * L, L)
            idx_v[sl] = idx_v[sl] + boff
        for c in range(n_ch):
            pltpu.async_copy(
                x_hbm.at[idx_v.at[pl.ds(c * CH, CH)]], rows_v, sem
            ).wait()
            pltpu.sync_copy(rows_v, out_hbm.at[pl.ds(base + c * CH, CH)])

    return _k(xf, perm_i32)


def kernel(x, perm):
    B, S, D = x.shape
    xf = x.reshape(B * S, D)
    out = _gather_rows(xf, perm.astype(jnp.int32), B, S, D)
    return out.reshape(B, S, D)


# speedup vs baseline: 2.8332x; 1.0184x over previous
import functools

import jax
import jax.numpy as jnp
from jax import lax
from jax.experimental import pallas as pl
from jax.experimental.pallas import tpu as pltpu
from jax.experimental.pallas import tpu_sc as plsc


def _gather_rows(xf, perm_i32, B, S, D):
    info = plsc.get_sparse_core_info()
    NC, NS, L = info.num_cores, info.num_subcores, info.num_lanes
    NW = NC * NS
    rows_total = B * S
    rows_per_w = rows_total // NW
    CH = 32
    n_ch = rows_per_w // CH

    mesh = plsc.VectorSubcoreMesh(core_axis_name="c", subcore_axis_name="s")

    @functools.partial(
        pl.kernel,
        mesh=mesh,
        out_type=jax.ShapeDtypeStruct((rows_total, D), jnp.float32),
        scratch_types=[
            pltpu.VMEM((rows_per_w,), jnp.int32),
            pltpu.VMEM((CH, D), jnp.float32),
            pltpu.VMEM((CH, D), jnp.float32),
            pltpu.SemaphoreType.DMA,
            pltpu.SemaphoreType.DMA,
        ],
    )
    def _k(x_hbm, perm_hbm, out_hbm, idx_v, buf0, buf1, sem_g, sem_s):
        wid = lax.axis_index("s") * NC + lax.axis_index("c")
        base = wid * rows_per_w
        seq0 = base % S
        boff = base - seq0
        pltpu.sync_copy(perm_hbm.at[pl.ds(seq0, rows_per_w)], idx_v)
        for j in range(rows_per_w // L):
            sl = pl.ds(j * L, L)
            idx_v[sl] = idx_v[sl] + boff

        bufs = (buf0, buf1)

        def gather_start(c):
            return pltpu.async_copy(
                x_hbm.at[idx_v.at[pl.ds(c * CH, CH)]], bufs[c % 2], sem_g
            )

        g = [None] * n_ch
        s = [None] * n_ch
        g[0] = gather_start(0)
        for c in range(n_ch):
            g[c].wait()
            s[c] = pltpu.async_copy(
                bufs[c % 2], out_hbm.at[pl.ds(base + c * CH, CH)], sem_s
            )
            if c + 1 < n_ch:
                if c >= 1:
                    s[c - 1].wait()
                g[c + 1] = gather_start(c + 1)
        if n_ch >= 2:
            s[n_ch - 2].wait()
        s[n_ch - 1].wait()

    return _k(xf, perm_i32)


def kernel(x, perm):
    B, S, D = x.shape
    xf = x.reshape(B * S, D)
    out = _gather_rows(xf, perm.astype(jnp.int32), B, S, D)
    return out.reshape(B, S, D)


# speedup vs baseline: 3.0132x; 1.0636x over previous
import functools

import jax
import jax.numpy as jnp
from jax import lax
from jax.experimental import pallas as pl
from jax.experimental.pallas import tpu as pltpu
from jax.experimental.pallas import tpu_sc as plsc


def _gather_rows(xf, perm_i32, B, S, D):
    info = plsc.get_sparse_core_info()
    NC, NS, L = info.num_cores, info.num_subcores, info.num_lanes
    NW = NC * NS
    rows_total = B * S
    rows_per_w = rows_total // NW
    CH = 16
    NBUF = 6
    n_ch = rows_per_w // CH

    mesh = plsc.VectorSubcoreMesh(core_axis_name="c", subcore_axis_name="s")

    @functools.partial(
        pl.kernel,
        mesh=mesh,
        out_type=jax.ShapeDtypeStruct((rows_total, D), jnp.float32),
        scratch_types=[
            pltpu.VMEM((rows_per_w,), jnp.int32),
        ] + [pltpu.VMEM((CH, D), jnp.float32) for _ in range(NBUF)] + [
            pltpu.SemaphoreType.DMA,
            pltpu.SemaphoreType.DMA,
        ],
    )
    def _k(x_hbm, perm_hbm, out_hbm, idx_v, *bufs_and_sems):
        bufs = bufs_and_sems[:NBUF]
        sem_g, sem_s = bufs_and_sems[NBUF:]
        wid = lax.axis_index("s") * NC + lax.axis_index("c")
        base = wid * rows_per_w
        seq0 = base % S
        boff = base - seq0
        pltpu.sync_copy(perm_hbm.at[pl.ds(seq0, rows_per_w)], idx_v)
        for j in range(rows_per_w // L):
            sl = pl.ds(j * L, L)
            idx_v[sl] = idx_v[sl] + boff

        def gather_start(c):
            return pltpu.async_copy(
                x_hbm.at[idx_v.at[pl.ds(c * CH, CH)]], bufs[c % NBUF], sem_g
            )

        def scatter_start(c):
            return pltpu.async_copy(
                bufs[c % NBUF], out_hbm.at[pl.ds(base + c * CH, CH)], sem_s
            )

        P = NBUF - 2
        g = [None] * n_ch
        s = [None] * n_ch
        for c in range(min(P, n_ch)):
            g[c] = gather_start(c)
        for c in range(n_ch):
            g[c].wait()
            s[c] = scatter_start(c)
            nxt = c + P
            if nxt < n_ch:
                prev = nxt - NBUF
                if prev >= 0:
                    s[prev].wait()
                g[nxt] = gather_start(nxt)
        for c in range(max(0, n_ch - P - 2), n_ch):
            s[c].wait()

    return _k(xf, perm_i32)


def kernel(x, perm):
    B, S, D = x.shape
    xf = x.reshape(B * S, D)
    out = _gather_rows(xf, perm.astype(jnp.int32), B, S, D)
    return out.reshape(B, S, D)
